# baseline (speedup 1.0000x reference)
import functools

import jax
import jax.numpy as jnp
from jax import lax
from jax.experimental import pallas as pl
from jax.experimental.pallas import tpu as pltpu
from jax.experimental.pallas import tpu_sc as plsc

EMB = 1000000
D = 64
SCALE = float(D) ** 0.5

NC = 2
NS = 16
NW = NC * NS


def _scale_body(w_ref, o_ref):
    o_ref[...] = w_ref[...] * SCALE


def _scale_table(weight):
    rows = 8000
    grid = EMB // rows
    return pl.pallas_call(
        _scale_body,
        grid=(grid,),
        in_specs=[pl.BlockSpec((rows, D), lambda i: (i, 0))],
        out_specs=pl.BlockSpec((rows, D), lambda i: (i, 0)),
        out_shape=jax.ShapeDtypeStruct((EMB, D), jnp.float32),
    )(weight)


def _make_gather(B, chunk):
    bpw = B // NW
    niter = bpw // chunk
    mesh = plsc.VectorSubcoreMesh(core_axis_name="c", subcore_axis_name="s")

    @functools.partial(
        pl.kernel,
        mesh=mesh,
        out_type=jax.ShapeDtypeStruct((B, D), jnp.float32),
        scratch_types=[
            pltpu.VMEM((chunk,), jnp.int32),
            pltpu.VMEM((chunk, D), jnp.float32),
            pltpu.SemaphoreType.DMA,
        ],
        compiler_params=pltpu.CompilerParams(use_tc_tiling_on_sc=False),
    )
    def gather_k(table_hbm, idx_hbm, out_hbm, idx_v, rows_v, sem):
        wid = lax.axis_index("s") * NC + lax.axis_index("c")
        wbase = wid * bpw

        def body(g, carry):
            base = wbase + g * chunk
            pltpu.sync_copy(idx_hbm.at[pl.ds(base, chunk)], idx_v)
            pltpu.async_copy(table_hbm.at[idx_v], rows_v, sem).wait()
            pltpu.sync_copy(rows_v, out_hbm.at[pl.ds(base, chunk)])
            return carry

        lax.fori_loop(0, niter, body, 0)

    return gather_k


@jax.jit
def kernel(x, weight):
    b0, b1 = x.shape
    B = b0 * b1
    xf = x.reshape(B).astype(jnp.int32)
    scaled = _scale_table(weight)
    out = _make_gather(B, 512)(scaled, xf)
    return out.reshape(b0, b1, D)


# speedup vs baseline: 1.2162x; 1.2162x over previous
import functools

import jax
import jax.numpy as jnp
from jax import lax
from jax.experimental import pallas as pl
from jax.experimental.pallas import tpu as pltpu
from jax.experimental.pallas import tpu_sc as plsc

D = 64
SCALE = float(D) ** 0.5

NC = 2
NS = 16
NW = NC * NS


def _scale_rows(rows_ref, chunk):
    rows_per_it = 8
    groups = D // 16

    def body(i, c):
        r0 = i * rows_per_it
        for rr in range(rows_per_it):
            for k in range(groups):
                sl = pl.ds(k * 16, 16)
                rows_ref[r0 + rr, sl] = rows_ref[r0 + rr, sl] * SCALE
        return c

    lax.fori_loop(0, chunk // rows_per_it, body, 0)


def _make_gather(B, chunk):
    bpw = B // NW
    niter = bpw // chunk
    assert niter % 2 == 0
    mesh = plsc.VectorSubcoreMesh(core_axis_name="c", subcore_axis_name="s")

    @functools.partial(
        pl.kernel,
        mesh=mesh,
        out_type=jax.ShapeDtypeStruct((B, D), jnp.float32),
        scratch_types=[
            pltpu.VMEM((chunk,), jnp.int32),
            pltpu.VMEM((chunk,), jnp.int32),
            pltpu.VMEM((chunk, D), jnp.float32),
            pltpu.VMEM((chunk, D), jnp.float32),
            pltpu.SemaphoreType.DMA,
            pltpu.SemaphoreType.DMA,
            pltpu.SemaphoreType.DMA,
            pltpu.SemaphoreType.DMA,
            pltpu.SemaphoreType.DMA,
            pltpu.SemaphoreType.DMA,
        ],
        compiler_params=pltpu.CompilerParams(use_tc_tiling_on_sc=False),
    )
    def gather_k(table_hbm, idx_hbm, out_hbm,
                 idx0, idx1, rows0, rows1,
                 isem0, isem1, gsem0, gsem1, ssem0, ssem1):
        wid = lax.axis_index("s") * NC + lax.axis_index("c")
        wbase = wid * bpw
        last = niter - 1

        idx_b = (idx0, idx1)
        rows_b = (rows0, rows1)
        isem_b = (isem0, isem1)
        gsem_b = (gsem0, gsem1)
        ssem_b = (ssem0, ssem1)

        def istart(g, s):
            base = wbase + jnp.minimum(g, last) * chunk
            pltpu.make_async_copy(
                idx_hbm.at[pl.ds(base, chunk)], idx_b[s], isem_b[s]).start()

        def iwait(s):
            pltpu.make_async_copy(
                idx_hbm.at[pl.ds(0, chunk)], idx_b[s], isem_b[s]).wait()

        def gstart(s):
            pltpu.make_async_copy(
                table_hbm.at[idx_b[s]], rows_b[s], gsem_b[s]).start()

        def gwait(s):
            pltpu.make_async_copy(
                table_hbm.at[idx_b[s]], rows_b[s], gsem_b[s]).wait()

        def sstart(g, s):
            base = wbase + g * chunk
            pltpu.make_async_copy(
                rows_b[s], out_hbm.at[pl.ds(base, chunk)], ssem_b[s]).start()

        def swait(s):
            pltpu.make_async_copy(
                rows_b[0], out_hbm.at[pl.ds(wbase, chunk)], ssem_b[s]).wait()

        istart(0, 0)
        iwait(0)
        gstart(0)
        istart(1, 1)

        def body(j, carry):
            g = 2 * j
            gwait(0)

            @pl.when(j > 0)
            def _():
                swait(1)

            iwait(1)
            gstart(1)
            _scale_rows(rows_b[0], chunk)
            sstart(g, 0)
            istart(g + 2, 0)

            g = 2 * j + 1
            gwait(1)
            swait(0)
            iwait(0)
            gstart(0)
            _scale_rows(rows_b[1], chunk)
            sstart(g, 1)
            istart(g + 2, 1)
            return carry

        lax.fori_loop(0, niter // 2, body, 0)

        swait(1)
        gwait(0)
        iwait(1)

    return gather_k


@jax.jit
def kernel(x, weight):
    b0, b1 = x.shape
    B = b0 * b1
    xf = x.reshape(B).astype(jnp.int32)
    out = _make_gather(B, 512)(weight, xf)
    return out.reshape(b0, b1, D)
